# baseline (speedup 1.0000x reference)
import jax
import jax.numpy as jnp
from jax import lax
from jax.experimental import pallas as pl
from jax.experimental.pallas import tpu as pltpu, tpu_sc as plsc

LENGTH = 64
BATCH = 64
NC = 2
NS = 16
NW = NC * NS
BPW = BATCH // NW
LSPLIT = 32
NEG = -3.0e38


def _run_levels(lo, hi, s_hbm, b0, b1,
                chS0a, chS1a, chS0b, chS1b,
                sbufa0, sbufb0, sbufa1, sbufb1,
                sema0, semb0, sema1, semb1):
    off = lo - 1

    def chunk(l, j, b, sb, sm):
        return pltpu.make_async_copy(
            s_hbm.at[l - 1 - off, pl.ds(j * 8, 8), b, :],
            sb.at[pl.ds(j * 8, 8), :], sm)

    def compute_level(l, sba, sbb):
        Lp = 64 - l
        nblk = (Lp + 15) // 16

        def blk_body(blk, _):
            pos0 = blk * 16
            neg = jnp.full((16,), NEG, jnp.float32)

            def step(k, carry):
                a0a, a1a, a0b, a1b, lo_, ro, nn = carry
                lk = lo_ + (64 * k)
                rk = ro - (63 * k)
                nk = nn + k
                svA = sba[nk, pl.ds(pos0, 16)]
                svB = sbb[nk, pl.ds(pos0, 16)]
                a0a = jnp.maximum(a0a, svA + chS0a[pl.ds(lk, 16)]
                                  + chS0a[pl.ds(rk, 16)])
                a1a = jnp.maximum(a1a, svA + chS1a[pl.ds(lk, 16)]
                                  + chS1a[pl.ds(rk, 16)])
                a0b = jnp.maximum(a0b, svB + chS0b[pl.ds(lk, 16)]
                                  + chS0b[pl.ds(rk, 16)])
                a1b = jnp.maximum(a1b, svB + chS1b[pl.ds(lk, 16)]
                                  + chS1b[pl.ds(rk, 16)])
                return a0a, a1a, a0b, a1b

            def n_body4(i, carry):
                for k in range(4):
                    out = step(k, carry)
                    carry = out + carry[4:]
                return out + (carry[4] + 256, carry[5] - 252, carry[6] + 4)

            def n_body1(n, carry):
                out = step(0, carry)
                return out + (carry[4] + 64, carry[5] - 63, carry[6] + 1)

            init = (neg, neg, neg, neg, pos0, (l - 1) * 64 + 1 + pos0, 0)
            carry = lax.fori_loop(0, l >> 2, n_body4, init)
            carry = lax.fori_loop(l & ~3, l, n_body1, carry)
            a0a, a1a, a0b, a1b = carry[:4]

            base = l * 64 + pos0
            chS0a[pl.ds(base, 16)] = a0a
            chS0b[pl.ds(base, 16)] = a0b
            chS1a[pl.ds(base, 16)] = a1a + chS1a[pl.ds(base, 16)]
            chS1b[pl.ds(base, 16)] = a1b + chS1b[pl.ds(base, 16)]
            return 0

        lax.fori_loop(0, nblk, blk_body, 0)

    def start(l, sba, sbb, sma, smb):
        def go(j, _):
            chunk(l, j, b0, sba, sma).start()
            chunk(l, j, b1, sbb, smb).start()
            return 0
        lax.fori_loop(0, (l + 7) >> 3, go, 0)

    def wait(l, sba, sbb, sma, smb):
        def go(j, _):
            chunk(l, j, b0, sba, sma).wait()
            chunk(l, j, b1, sbb, smb).wait()
            return 0
        lax.fori_loop(0, (l + 7) >> 3, go, 0)

    bufs = ((sbufa0, sbufb0, sema0, semb0),
            (sbufa1, sbufb1, sema1, semb1))
    start(lo, *bufs[0])
    npairs = (hi - lo - 1) // 2

    def pair_body(i, _):
        l = lo + 2 * i
        wait(l, *bufs[0])
        start(l + 1, *bufs[1])
        compute_level(l, bufs[0][0], bufs[0][1])
        wait(l + 1, *bufs[1])
        start(l + 2, *bufs[0])
        compute_level(l + 1, bufs[1][0], bufs[1][1])
        return 0

    lax.fori_loop(0, npairs, pair_body, 0)
    for r in range(lo + 2 * npairs, hi + 1):
        par = (r - lo) & 1
        wait(r, *bufs[par])
        if r < hi:
            start(r + 1, *bufs[1 - par])
        compute_level(r, bufs[par][0], bufs[par][1])


def _decode_cons(cbuf, iota):
    row8 = iota & 7
    pv = plsc.load_gather(cbuf, [row8, jnp.zeros((16,), jnp.int32)])
    sz = plsc.load_gather(cbuf, [row8, jnp.full((16,), 1, jnp.int32)])
    lv = sz - 1
    valid = (lv > 0) & (iota < 8)
    return pv, lv, valid


def _sc_body_a(s_hbm, cons_hbm, state_out,
               chS0a, chS1a, chS0b, chS1b,
               sbufa0, sbufb0, sbufa1, sbufb1,
               cbufa, cbufb,
               sema0, semb0, sema1, semb1):
    wid = lax.axis_index("s") * NC + lax.axis_index("c")
    b0 = wid * BPW
    b1 = b0 + 1
    iota = lax.iota(jnp.int32, 16)
    zeros = jnp.zeros((16,), jnp.float32)

    def zero_s(r, _):
        for ref in (chS0a, chS1a, chS0b, chS1b):
            ref[pl.ds(r * 16, 16)] = zeros
        return 0
    lax.fori_loop(0, 64 * 64 // 16, zero_s, 0)

    pltpu.sync_copy(cons_hbm.at[b0], cbufa)
    pltpu.sync_copy(cons_hbm.at[b1], cbufb)
    thousand = jnp.full((16,), 1000.0, jnp.float32)
    for cbuf, chS1 in ((cbufa, chS1a), (cbufb, chS1b)):
        pv, lv, valid = _decode_cons(cbuf, iota)
        lvc = jnp.where(valid, lv, 0)
        pvc = jnp.where(valid, pv, 0)
        plsc.store_scatter(chS1, [lvc * 64 + pvc], thousand, mask=valid)

    _run_levels(1, LSPLIT, s_hbm, b0, b1,
                chS0a, chS1a, chS0b, chS1b,
                sbufa0, sbufb0, sbufa1, sbufb1,
                sema0, semb0, sema1, semb1)

    for c, ref in enumerate((chS0a, chS1a, chS0b, chS1b)):
        pltpu.sync_copy(ref, state_out.at[wid, c])


def _sc_body_b(s_hbm, cons_hbm, state_in, out_hbm,
               chS0a, chS1a, chS0b, chS1b,
               sbufa0, sbufb0, sbufa1, sbufb1,
               cbufa, cbufb, ostage,
               sema0, semb0, sema1, semb1):
    wid = lax.axis_index("s") * NC + lax.axis_index("c")
    b0 = wid * BPW
    b1 = b0 + 1
    iota = lax.iota(jnp.int32, 16)
    zeros = jnp.zeros((16,), jnp.float32)

    for c, ref in enumerate((chS0a, chS1a, chS0b, chS1b)):
        pltpu.sync_copy(state_in.at[wid, c], ref)

    pltpu.sync_copy(cons_hbm.at[b0], cbufa)
    pltpu.sync_copy(cons_hbm.at[b1], cbufb)
    subs = []
    for cbuf in (cbufa, cbufb):
        _, _, valid = _decode_cons(cbuf, iota)
        cnt = plsc.all_reduce_population_count(valid)
        subs.append(cnt.astype(jnp.float32) * 1000.0)

    _run_levels(LSPLIT + 1, 63, s_hbm, b0, b1,
                chS0a, chS1a, chS0b, chS1b,
                sbufa0, sbufb0, sbufa1, sbufb1,
                sema0, semb0, sema1, semb1)

    root = jnp.full((16,), 63 * 64, jnp.int32)
    p0 = plsc.load_gather(chS0a, [root])
    p1 = plsc.load_gather(chS0b, [root])
    c0 = plsc.load_gather(chS1a, [root]) - subs[0]
    c1 = plsc.load_gather(chS1b, [root]) - subs[1]
    ov = jnp.where(iota == 0, p0, zeros)
    ov = jnp.where(iota == 1, p1, ov)
    ov = jnp.where(iota == 2, c0, ov)
    ov = jnp.where(iota == 3, c1, ov)
    ostage[...] = ov
    pltpu.sync_copy(ostage, out_hbm.at[wid])


def _loss_body(x_ref, o_ref):
    x = x_ref[...]
    p = x[:, 0:2]
    c = x[:, 2:4]
    diff = p - c
    mask = jnp.where(jnp.abs(diff) < 0.001, 0.0, 1.0)
    hinge = jnp.maximum(1.0 + diff, 0.0) * mask
    ms = jnp.sum(mask)
    hs = jnp.sum(hinge)
    o_ref[0, 0] = jnp.where(ms > 0.1, hs / jnp.maximum(ms, 1e-9), hs)


def kernel(score_components, constraints):
    sc4 = jnp.transpose(score_components[..., 0], (0, 3, 2, 1))
    cons = constraints.astype(jnp.int32)
    sc_a = sc4[:LSPLIT]
    sc_b = sc4[LSPLIT:]

    mesh = plsc.VectorSubcoreMesh(core_axis_name="c", subcore_axis_name="s")
    f32 = jnp.float32
    scratch = [
        pltpu.VMEM((64 * 64,), f32), pltpu.VMEM((64 * 64,), f32),
        pltpu.VMEM((64 * 64,), f32), pltpu.VMEM((64 * 64,), f32),
        pltpu.VMEM((64, 63), f32), pltpu.VMEM((64, 63), f32),
        pltpu.VMEM((64, 63), f32), pltpu.VMEM((64, 63), f32),
        pltpu.VMEM((8, 2), jnp.int32), pltpu.VMEM((8, 2), jnp.int32),
    ]
    sems = [pltpu.SemaphoreType.DMA] * 4

    state = pl.kernel(
        _sc_body_a,
        out_type=jax.ShapeDtypeStruct((NW, 4, 64 * 64), f32),
        mesh=mesh,
        scratch_types=scratch + sems,
        compiler_params=pltpu.CompilerParams(needs_layout_passes=False),
    )(sc_a, cons)

    stats = pl.kernel(
        _sc_body_b,
        out_type=jax.ShapeDtypeStruct((NW, 16), f32),
        mesh=mesh,
        scratch_types=scratch + [pltpu.VMEM((16,), f32)] + sems,
        compiler_params=pltpu.CompilerParams(needs_layout_passes=False),
    )(sc_b, cons, state)

    loss = pl.pallas_call(
        _loss_body,
        out_shape=jax.ShapeDtypeStruct((1, 1), f32),
        out_specs=pl.BlockSpec(memory_space=pltpu.SMEM),
    )(stats)
    return loss[0, 0]


# speedup vs baseline: 1.4890x; 1.4890x over previous
import jax
import jax.numpy as jnp
from jax import lax
from jax.experimental import pallas as pl
from jax.experimental.pallas import tpu as pltpu, tpu_sc as plsc

LENGTH = 64
BATCH = 64
NC = 2
NS = 16
NW = NC * NS
BPW = BATCH // NW
ETS = 128
NEG = -3.0e38


def _sc_body(s_hbm, cons_hbm, out_hbm,
             chS0a, chS1a, chS0b, chS1b,
             sbufa0, sbufb0, sbufa1, sbufb1,
             cbufa, cbufb, ostage,
             sema0, semb0, sema1, semb1):
    wid = lax.axis_index("s") * NC + lax.axis_index("c")
    b0 = wid * BPW
    b1 = b0 + 1

    iota = lax.iota(jnp.int32, 16)
    zeros = jnp.zeros((16,), jnp.float32)
    izero16 = jnp.zeros((16,), jnp.int32)

    def zero_s(r, _):
        for ref in (chS0a, chS1a, chS0b, chS1b):
            ref[pl.ds(r * 16, 16)] = zeros
        return 0
    lax.fori_loop(0, 64 * 64 // 16, zero_s, 0)

    pltpu.sync_copy(cons_hbm.at[b0], cbufa)
    pltpu.sync_copy(cons_hbm.at[b1], cbufb)
    thousand = jnp.full((16,), 1000.0, jnp.float32)
    subs = []
    for cbuf, chS1 in ((cbufa, chS1a), (cbufb, chS1b)):
        row8 = iota & 7
        pv = plsc.load_gather(cbuf, [row8, jnp.zeros((16,), jnp.int32)])
        sz = plsc.load_gather(cbuf, [row8, jnp.full((16,), 1, jnp.int32)])
        lv = sz - 1
        valid = (lv > 0) & (iota < 8)
        lvc = jnp.where(valid, lv, 0)
        pvc = jnp.where(valid, pv, 0)
        plsc.store_scatter(chS1, [lvc * 64 + pvc], thousand, mask=valid)
        cnt = plsc.all_reduce_population_count(valid)
        subs.append(cnt.astype(jnp.float32) * 1000.0)

    def chunk(l, j, b, sb, sm):
        return pltpu.make_async_copy(s_hbm.at[l - 1, pl.ds(j * 8, 8), b, :],
                                     sb.at[pl.ds(j * 8, 8), :], sm)

    def compute_level(l, sba, sbb):
        Lp = 64 - l
        nblk = (Lp + 15) // 16

        def blk_body(blk, _):
            pos0 = blk * 16
            neg = jnp.full((16,), NEG, jnp.float32)

            def step(k, carry):
                a0a, a1a, a0b, a1b, lo, ro, nn = carry
                lk = lo + (64 * k)
                rk = ro - (63 * k)
                nk = nn + k
                svA = sba[nk, pl.ds(pos0, 16)]
                svB = sbb[nk, pl.ds(pos0, 16)]
                a0a = jnp.maximum(a0a, svA + chS0a[pl.ds(lk, 16)]
                                  + chS0a[pl.ds(rk, 16)])
                a1a = jnp.maximum(a1a, svA + chS1a[pl.ds(lk, 16)]
                                  + chS1a[pl.ds(rk, 16)])
                a0b = jnp.maximum(a0b, svB + chS0b[pl.ds(lk, 16)]
                                  + chS0b[pl.ds(rk, 16)])
                a1b = jnp.maximum(a1b, svB + chS1b[pl.ds(lk, 16)]
                                  + chS1b[pl.ds(rk, 16)])
                return a0a, a1a, a0b, a1b

            def n_body4(i, carry):
                for k in range(4):
                    out = step(k, carry)
                    carry = out + carry[4:]
                return out + (carry[4] + 256, carry[5] - 252, carry[6] + 4)

            def n_body1(n, carry):
                out = step(0, carry)
                return out + (carry[4] + 64, carry[5] - 63, carry[6] + 1)

            init = (neg, neg, neg, neg, pos0, (l - 1) * 64 + 1 + pos0, 0)
            carry = lax.fori_loop(0, l >> 2, n_body4, init)
            carry = lax.fori_loop(l & ~3, l, n_body1, carry)
            a0a, a1a, a0b, a1b = carry[:4]

            base = l * 64 + pos0
            chS0a[pl.ds(base, 16)] = a0a
            chS0b[pl.ds(base, 16)] = a0b
            chS1a[pl.ds(base, 16)] = a1a + chS1a[pl.ds(base, 16)]
            chS1b[pl.ds(base, 16)] = a1b + chS1b[pl.ds(base, 16)]
            return 0

        lax.fori_loop(0, nblk, blk_body, 0)

    def start(l, sba, sbb, sma, smb):
        def go(j, _):
            chunk(l, j, b0, sba, sma).start()
            chunk(l, j, b1, sbb, smb).start()
            return 0
        lax.fori_loop(0, (l + 7) >> 3, go, 0)

    def wait(l, sba, sbb, sma, smb):
        def go(j, _):
            chunk(l, j, b0, sba, sma).wait()
            chunk(l, j, b1, sbb, smb).wait()
            return 0
        lax.fori_loop(0, (l + 7) >> 3, go, 0)

    buf0 = (sbufa0, sbufb0, sema0, semb0)
    buf1 = (sbufa1, sbufb1, sema1, semb1)
    start(1, *buf0)

    def pair_body(i, _):
        l = 2 * i + 1
        wait(l, *buf0)
        start(l + 1, *buf1)
        compute_level(l, sbufa0, sbufb0)
        wait(l + 1, *buf1)
        start(l + 2, *buf0)
        compute_level(l + 1, sbufa1, sbufb1)
        return 0

    lax.fori_loop(0, 31, pair_body, 0)
    wait(63, *buf0)
    compute_level(63, sbufa0, sbufb0)

    root = jnp.full((16,), 63 * 64, jnp.int32)
    p0 = plsc.load_gather(chS0a, [root])
    p1 = plsc.load_gather(chS0b, [root])
    c0 = plsc.load_gather(chS1a, [root]) - subs[0]
    c1 = plsc.load_gather(chS1b, [root]) - subs[1]
    ov = jnp.where(iota == 0, p0, zeros)
    ov = jnp.where(iota == 1, p1, ov)
    ov = jnp.where(iota == 2, c0, ov)
    ov = jnp.where(iota == 3, c1, ov)
    ostage[...] = ov
    pltpu.sync_copy(ostage, out_hbm.at[wid])


def _loss_body(x_ref, o_ref):
    x = x_ref[...]
    p = x[:, 0:2]
    c = x[:, 2:4]
    diff = p - c
    mask = jnp.where(jnp.abs(diff) < 0.001, 0.0, 1.0)
    hinge = jnp.maximum(1.0 + diff, 0.0) * mask
    ms = jnp.sum(mask)
    hs = jnp.sum(hinge)
    o_ref[0, 0] = jnp.where(ms > 0.1, hs / jnp.maximum(ms, 1e-9), hs)


def kernel(score_components, constraints):
    sc4 = jnp.transpose(score_components[..., 0], (0, 3, 2, 1))
    cons = constraints.astype(jnp.int32)

    mesh = plsc.VectorSubcoreMesh(core_axis_name="c", subcore_axis_name="s")
    f32 = jnp.float32
    stats = pl.kernel(
        _sc_body,
        out_type=jax.ShapeDtypeStruct((NW, 16), f32),
        mesh=mesh,
        scratch_types=[
            pltpu.VMEM((64 * 64,), f32), pltpu.VMEM((64 * 64,), f32),
            pltpu.VMEM((64 * 64,), f32), pltpu.VMEM((64 * 64,), f32),
            pltpu.VMEM((64, 63), f32), pltpu.VMEM((64, 63), f32),
            pltpu.VMEM((64, 63), f32), pltpu.VMEM((64, 63), f32),
            pltpu.VMEM((8, 2), jnp.int32), pltpu.VMEM((8, 2), jnp.int32),
            pltpu.VMEM((16,), f32),
            pltpu.SemaphoreType.DMA, pltpu.SemaphoreType.DMA,
            pltpu.SemaphoreType.DMA, pltpu.SemaphoreType.DMA,
        ],
        compiler_params=pltpu.CompilerParams(needs_layout_passes=False),
    )(sc4, cons)

    loss = pl.pallas_call(
        _loss_body,
        out_shape=jax.ShapeDtypeStruct((1, 1), f32),
        out_specs=pl.BlockSpec(memory_space=pltpu.SMEM),
    )(stats)
    return loss[0, 0]
